# baseline (speedup 1.0000x reference)
import functools

import jax
import jax.numpy as jnp
from jax import lax
from jax.experimental import pallas as pl
from jax.experimental.pallas import tpu as pltpu
from jax.experimental.pallas import tpu_sc as plsc

B = 16
H = W = 384
NPTS = 512
L = 16
HALF = H // 2
VECS = NPTS // L

_mesh = plsc.VectorSubcoreMesh(core_axis_name="c", subcore_axis_name="s")


@functools.partial(
    pl.kernel,
    mesh=_mesh,
    out_type=jax.ShapeDtypeStruct((2, B, L), jnp.float32),
    scratch_types=[
        pltpu.VMEM((3, NPTS), jnp.float32),
        pltpu.VMEM((HALF, W), jnp.float32),
        pltpu.VMEM((L,), jnp.float32),
        pltpu.SemaphoreType.DMA,
    ],
    compiler_params=pltpu.CompilerParams(needs_layout_passes=False),
)
def _depth_loss_kernel(
    img_hbm, rd_hbm, out_hbm,
    rd_v, img_v, part_v, sem,
):
    b = lax.axis_index("s")
    h = lax.axis_index("c")

    img2 = img_hbm.reshape(B * H, W)
    Q = HALF // 4
    row0 = h * HALF
    cps = [pltpu.async_copy(rd_hbm.at[:, b, :], rd_v, sem)]
    for q in range(4):
        cps.append(
            pltpu.async_copy(
                img2.at[pl.ds(b * H + row0 + q * Q, Q), :],
                img_v.at[pl.ds(q * Q, Q), :],
                sem,
            )
        )
    cps[0].wait()

    def make_step(lo):
        def step(v, carry):
            acc, cnt = carry
            for u in range(4):
                sl = pl.ds((v * 4 + u) * L, L)
                ri = rd_v[0, sl].astype(jnp.int32)
                ci = rd_v[1, sl].astype(jnp.int32)
                d = rd_v[2, sl]
                rl = ri - row0 - lo
                inq = (rl >= 0) & (rl < Q)
                m = inq & (d > 0.0)
                g = plsc.load_gather(
                    img_v, [jnp.where(inq, rl + lo, 0), ci]
                )
                acc = acc + jnp.where(m, jnp.abs(g - d), 0.0)
                cnt = cnt + jnp.where(m, 1.0, 0.0)
            return acc, cnt
        return step

    acc = jnp.zeros((L,), jnp.float32)
    cnt = jnp.zeros((L,), jnp.float32)
    for q in range(4):
        cps[q + 1].wait()
        acc, cnt = lax.fori_loop(0, VECS // 4, make_step(q * Q), (acc, cnt))

    lane = lax.iota(jnp.int32, L)
    sv = jnp.full((L,), jnp.sum(acc), jnp.float32)
    cv = jnp.full((L,), jnp.sum(cnt), jnp.float32)
    part_v[...] = jnp.where(lane == 0, sv, jnp.where(lane == 1, cv, 0.0))
    pltpu.sync_copy(part_v, out_hbm.at[h, b])


@jax.jit
def kernel(output, rdepth):
    res = _depth_loss_kernel(output, rdepth.transpose(2, 0, 1))
    tot = jnp.sum(res, axis=(0, 1))
    s = tot[0]
    c = tot[1]
    return jnp.where(c > 0.0, s / jnp.maximum(c, 1.0), jnp.float32(0.0))


# speedup vs baseline: 1.0257x; 1.0257x over previous
import functools

import jax
import jax.numpy as jnp
from jax import lax
from jax.experimental import pallas as pl
from jax.experimental.pallas import tpu as pltpu
from jax.experimental.pallas import tpu_sc as plsc

B = 16
H = W = 384
NPTS = 512
L = 16
HALF = H // 2
VECS = NPTS // L

_mesh = plsc.VectorSubcoreMesh(core_axis_name="c", subcore_axis_name="s")


@functools.partial(
    pl.kernel,
    mesh=_mesh,
    out_type=jax.ShapeDtypeStruct((2, B, L), jnp.float32),
    scratch_types=[
        pltpu.VMEM((3, NPTS), jnp.float32),
        pltpu.VMEM((HALF, W), jnp.float32),
        pltpu.VMEM((L,), jnp.float32),
        pltpu.SemaphoreType.DMA,
    ],
    compiler_params=pltpu.CompilerParams(needs_layout_passes=False),
)
def _depth_loss_kernel(
    img_hbm, rd_hbm, out_hbm,
    rd_v, img_v, part_v, sem,
):
    b = lax.axis_index("s")
    h = lax.axis_index("c")

    img2 = img_hbm.reshape(B * H, W)
    Q = HALF // 2
    row0 = h * HALF
    cps = [pltpu.async_copy(rd_hbm.at[:, b, :], rd_v, sem)]
    for q in range(2):
        cps.append(
            pltpu.async_copy(
                img2.at[pl.ds(b * H + row0 + q * Q, Q), :],
                img_v.at[pl.ds(q * Q, Q), :],
                sem,
            )
        )
    cps[0].wait()

    def make_step(lo):
        def step(v, carry):
            acc, cnt = carry
            for u in range(4):
                sl = pl.ds((v * 4 + u) * L, L)
                ri = rd_v[0, sl].astype(jnp.int32)
                ci = rd_v[1, sl].astype(jnp.int32)
                d = rd_v[2, sl]
                rl = ri - row0 - lo
                inq = (rl >= 0) & (rl < Q)
                m = inq & (d > 0.0)
                g = plsc.load_gather(
                    img_v, [jnp.where(inq, rl + lo, 0), ci]
                )
                acc = acc + jnp.where(m, jnp.abs(g - d), 0.0)
                cnt = cnt + jnp.where(m, 1.0, 0.0)
            return acc, cnt
        return step

    acc = jnp.zeros((L,), jnp.float32)
    cnt = jnp.zeros((L,), jnp.float32)
    for q in range(2):
        cps[q + 1].wait()
        acc, cnt = lax.fori_loop(0, VECS // 4, make_step(q * Q), (acc, cnt))

    lane = lax.iota(jnp.int32, L)
    sv = jnp.full((L,), jnp.sum(acc), jnp.float32)
    cv = jnp.full((L,), jnp.sum(cnt), jnp.float32)
    part_v[...] = jnp.where(lane == 0, sv, jnp.where(lane == 1, cv, 0.0))
    pltpu.sync_copy(part_v, out_hbm.at[h, b])


@jax.jit
def kernel(output, rdepth):
    res = _depth_loss_kernel(output, rdepth.transpose(2, 0, 1))
    tot = jnp.sum(res, axis=(0, 1))
    s = tot[0]
    c = tot[1]
    return jnp.where(c > 0.0, s / jnp.maximum(c, 1.0), jnp.float32(0.0))


# speedup vs baseline: 1.0426x; 1.0165x over previous
import functools

import jax
import jax.numpy as jnp
from jax import lax
from jax.experimental import pallas as pl
from jax.experimental.pallas import tpu as pltpu
from jax.experimental.pallas import tpu_sc as plsc

B = 16
H = W = 384
NPTS = 512
L = 16
HALF = H // 2
VECS = NPTS // L

_mesh = plsc.VectorSubcoreMesh(core_axis_name="c", subcore_axis_name="s")


@functools.partial(
    pl.kernel,
    mesh=_mesh,
    out_type=jax.ShapeDtypeStruct((2, B, L), jnp.float32),
    scratch_types=[
        pltpu.VMEM((3, NPTS), jnp.float32),
        pltpu.VMEM((HALF, W), jnp.float32),
        pltpu.VMEM((L,), jnp.float32),
        pltpu.SemaphoreType.DMA,
    ],
    compiler_params=pltpu.CompilerParams(needs_layout_passes=False),
)
def _depth_loss_kernel(
    img_hbm, rd_hbm, out_hbm,
    rd_v, img_v, part_v, sem,
):
    b = lax.axis_index("s")
    h = lax.axis_index("c")

    img2 = img_hbm.reshape(B * H, W)
    row0 = h * HALF
    cp0 = pltpu.async_copy(
        img2.at[pl.ds(b * H + row0, HALF), :], img_v, sem
    )
    cp1 = pltpu.async_copy(rd_hbm.at[:, b, :], rd_v, sem)
    cp0.wait()
    cp1.wait()

    def step(v, carry):
        acc, cnt = carry
        for u in range(4):
            sl = pl.ds((v * 4 + u) * L, L)
            ri = rd_v[0, sl].astype(jnp.int32)
            ci = rd_v[1, sl].astype(jnp.int32)
            d = rd_v[2, sl]
            rl = ri - row0
            inh = (rl >= 0) & (rl < HALF)
            m = inh & (d > 0.0)
            g = plsc.load_gather(img_v, [jnp.where(inh, rl, 0), ci])
            acc = acc + jnp.where(m, jnp.abs(g - d), 0.0)
            cnt = cnt + jnp.where(m, 1.0, 0.0)
        return acc, cnt

    acc = jnp.zeros((L,), jnp.float32)
    cnt = jnp.zeros((L,), jnp.float32)
    acc, cnt = lax.fori_loop(0, VECS // 4, step, (acc, cnt))

    lane = lax.iota(jnp.int32, L)
    sv = jnp.full((L,), jnp.sum(acc), jnp.float32)
    cv = jnp.full((L,), jnp.sum(cnt), jnp.float32)
    part_v[...] = jnp.where(lane == 0, sv, jnp.where(lane == 1, cv, 0.0))
    pltpu.sync_copy(part_v, out_hbm.at[h, b])


@jax.jit
def kernel(output, rdepth):
    res = _depth_loss_kernel(output, rdepth.transpose(2, 0, 1))
    tot = jnp.sum(res, axis=(0, 1))
    s = tot[0]
    c = tot[1]
    return jnp.where(c > 0.0, s / jnp.maximum(c, 1.0), jnp.float32(0.0))
